# baseline (speedup 1.0000x reference)
import jax
import jax.numpy as jnp
from jax import lax
from jax.experimental import pallas as pl
from jax.experimental.pallas import tpu as pltpu
from jax.experimental.pallas import tpu_sc as plsc

N = 10000
E = 320000
D_LAT = 128
D_EDGE = 16
D_HID = 128
D_OUT = 64

NC = 2
NS = 16
NW = NC * NS
L = 16

CHUNK = 128
HCH = CHUNK // 2
NCH = E // CHUNK
ITERS = (NCH + NW - 1) // NW
ITERS2 = (NCH + NS - 1) // NS
N_PAD = 10240
NHALF = N_PAD // 2
TROWS_T = NHALF // NS
ROWS_PER_TILE = N_PAD // NS
ZROWS = CHUNK


def _pre_body(hn_ref, wa_ref, wb_ref, b1_ref, p1_ref, p2_ref):
    h = hn_ref[...]
    p1 = jnp.dot(h, wa_ref[...], preferred_element_type=jnp.float32) + b1_ref[...]
    p2 = jnp.dot(h, wb_ref[...], preferred_element_type=jnp.float32)
    p1_ref[...] = p1.astype(jnp.bfloat16)
    p2_ref[...] = p2.astype(jnp.bfloat16)


def _precompute(hn, we1a, we1b, be1):
    blk = 1000
    return pl.pallas_call(
        _pre_body,
        grid=(N // blk,),
        in_specs=[
            pl.BlockSpec((blk, D_LAT), lambda i: (i, 0)),
            pl.BlockSpec((D_LAT, D_HID), lambda i: (0, 0)),
            pl.BlockSpec((D_LAT, D_HID), lambda i: (0, 0)),
            pl.BlockSpec((1, D_HID), lambda i: (0, 0)),
        ],
        out_specs=[
            pl.BlockSpec((blk, D_HID), lambda i: (i, 0)),
            pl.BlockSpec((blk, D_HID), lambda i: (i, 0)),
        ],
        out_shape=[
            jax.ShapeDtypeStruct((N, D_HID), jnp.bfloat16),
            jax.ShapeDtypeStruct((N, D_HID), jnp.bfloat16),
        ],
    )(hn, we1a, we1b, be1)


def _gather_body(src_hbm, dst_hbm, t_hbm, g_hbm,
                 i1a, i2a, b1a, b2a, bga, sga,
                 i1b, i2b, b1b, b2b, bgb, sgb, t_sh):
    cid = lax.axis_index("c")
    sid = lax.axis_index("s")
    base_node = cid * NHALF

    slots = ((i1a, i2a, b1a, b2a, bga, sga), (i1b, i2b, b1b, b2b, bgb, sgb))

    for r0, sz in ((0, 128), (128, 128), (256, 64)):
        pltpu.sync_copy(t_hbm.at[pl.ds(base_node + sid * TROWS_T + r0, sz)],
                        b1a.at[pl.ds(0, sz)])
        pltpu.sync_copy(b1a.at[pl.ds(0, sz)],
                        t_sh.at[pl.ds(sid * TROWS_T + r0, sz)])

    @pl.when(sid == 0)
    def _():
        zeros = jnp.zeros((L,), jnp.float32)

        def zrow(r, carry):
            for j in range(D_HID // L):
                b1a[r, pl.ds(j * L, L)] = zeros
            return carry

        lax.fori_loop(0, 8, zrow, 0)
        pltpu.sync_copy(b1a.at[pl.ds(0, 8)], t_sh.at[pl.ds(NHALF, 8)])

    plsc.subcore_barrier()

    def localize(idx):
        for k in range(CHUNK // L):
            sl = pl.ds(k * L, L)
            loc = idx[sl] - base_node
            ok = (loc >= 0) & (loc < NHALF)
            idx[sl] = jnp.where(ok, loc, NHALF)

    def start(i, slot):
        i1, i2, b1, b2, bg, sg = slot
        c = i * NS + sid

        @pl.when(c < NCH)
        def _():
            base = c * CHUNK
            pltpu.sync_copy(src_hbm.at[pl.ds(base, CHUNK)], i1)
            pltpu.sync_copy(dst_hbm.at[pl.ds(base, CHUNK)], i2)
            localize(i1)
            localize(i2)
            pltpu.async_copy(t_sh.at[i1], b1, sg)
            pltpu.async_copy(t_sh.at[i2], b2, sg)

    def finish(i, slot):
        i1, i2, b1, b2, bg, sg = slot
        c = i * NS + sid

        @pl.when(c < NCH)
        def _():
            pltpu.make_async_copy(t_sh.at[i1], b1, sg).wait()
            pltpu.make_async_copy(t_sh.at[i2], b2, sg).wait()

            M = jnp.uint32(0xFFFF0000)
            R = jnp.uint32(0x8000)

            def packadd(w1, w2):
                u1 = jax.lax.bitcast_convert_type(w1, jnp.uint32)
                u2 = jax.lax.bitcast_convert_type(w2, jnp.uint32)
                lo = (jax.lax.bitcast_convert_type(u1 << 16, jnp.float32)
                      + jax.lax.bitcast_convert_type(u2 << 16, jnp.float32))
                hi = (jax.lax.bitcast_convert_type(u1 & M, jnp.float32)
                      + jax.lax.bitcast_convert_type(u2 & M, jnp.float32))
                ul = (jax.lax.bitcast_convert_type(lo, jnp.uint32) + R) >> 16
                uh = (jax.lax.bitcast_convert_type(hi, jnp.uint32) + R) & M
                return jax.lax.bitcast_convert_type(ul | uh, jnp.float32)

            def pairrow(t, rcarry):
                rb = t + HCH
                for j in range(64 // L):
                    cs = pl.ds(j * L, L)
                    cs2 = pl.ds(64 + j * L, L)
                    bg[t, cs] = packadd(b1[t, cs], b2[t, cs2])
                    bg[t, cs2] = packadd(b1[rb, cs], b2[rb, cs2])
                return rcarry

            lax.fori_loop(0, HCH, pairrow, 0)
            pltpu.sync_copy(bg, g_hbm.at[cid, pl.ds(c * HCH, HCH)])

    for b in range(2):
        start(b, slots[b])

    def pair(p, carry):
        for b in range(2):
            i = p * 2 + b
            finish(i, slots[b])
            start(i + 2, slots[b])
        return carry

    lax.fori_loop(0, (ITERS2 + 1) // 2, pair, 0)


def _gather(src, dst, tpk):
    slot = [
        pltpu.VMEM((CHUNK,), jnp.int32),
        pltpu.VMEM((CHUNK,), jnp.int32),
        pltpu.VMEM((CHUNK, D_HID), jnp.float32),
        pltpu.VMEM((CHUNK, D_HID), jnp.float32),
        pltpu.VMEM((HCH, D_HID), jnp.float32),
        pltpu.SemaphoreType.DMA,
    ]
    fn = pl.kernel(
        _gather_body,
        out_type=jax.ShapeDtypeStruct((NC, E // 2, D_HID), jnp.float32),
        mesh=plsc.VectorSubcoreMesh(
            core_axis_name="c", subcore_axis_name="s",
            num_cores=NC, num_subcores=NS),
        scratch_types=slot + slot + [
            pltpu.VMEM_SHARED((NHALF + 8, D_HID), jnp.float32),
        ],
    )
    return fn(src, dst, tpk)


KPB = 10
BLK_E = KPB * CHUNK
BLK_G = KPB * HCH


def _unpack(g):
    u = jax.lax.bitcast_convert_type(g, jnp.uint32)
    lo = jax.lax.bitcast_convert_type(u << 16, jnp.float32)
    hi = jax.lax.bitcast_convert_type(u & jnp.uint32(0xFFFF0000), jnp.float32)
    return lo, hi


def _edge_body(g0_ref, g1_ref, he_ref, wc_ref, w2_ref, b2_ref, out_ref):
    lo0, hi0 = _unpack(g0_ref[...])
    lo1, hi1 = _unpack(g1_ref[...])
    lo = lo0 + lo1
    hi = hi0 + hi1
    xa = jnp.concatenate([lo[:, :64], hi[:, :64]], axis=1)
    xb = jnp.concatenate([lo[:, 64:], hi[:, 64:]], axis=1)
    x = jnp.concatenate([xa, xb], axis=0)
    he_blk = he_ref[...]
    hea = jnp.concatenate(
        [he_blk[CHUNK * k:CHUNK * k + HCH] for k in range(KPB)], axis=0)
    heb = jnp.concatenate(
        [he_blk[CHUNK * k + HCH:CHUNK * (k + 1)] for k in range(KPB)], axis=0)
    he_sel = jnp.concatenate([hea, heb], axis=0)
    x = x + jnp.dot(he_sel, wc_ref[...], preferred_element_type=jnp.float32)
    s = x * jax.nn.sigmoid(x)
    o = jnp.dot(s, w2_ref[...], preferred_element_type=jnp.float32) + b2_ref[...]
    for k in range(KPB):
        out_ref[pl.ds(CHUNK * k, HCH), :] = o[HCH * k:HCH * (k + 1)]
        out_ref[pl.ds(CHUNK * k + HCH, HCH), :] = o[BLK_G + HCH * k:BLK_G + HCH * (k + 1)]


def _edge_mlp(g0, g1, he, we1c, we2, be2):
    return pl.pallas_call(
        _edge_body,
        grid=(E // BLK_E,),
        in_specs=[
            pl.BlockSpec((BLK_G, D_HID), lambda i: (i, 0)),
            pl.BlockSpec((BLK_G, D_HID), lambda i: (i, 0)),
            pl.BlockSpec((BLK_E, D_EDGE), lambda i: (i, 0)),
            pl.BlockSpec((D_EDGE, D_HID), lambda i: (0, 0)),
            pl.BlockSpec((D_HID, D_HID), lambda i: (0, 0)),
            pl.BlockSpec((1, D_HID), lambda i: (0, 0)),
        ],
        out_specs=pl.BlockSpec((BLK_E, D_HID), lambda i: (i, 0)),
        out_shape=jax.ShapeDtypeStruct((E, D_HID), jnp.float32),
    )(g0, g1, he, we1c, we2, be2)


def _scatter_body(dst_hbm, hen_hbm, agg_hbm,
                  ia, bufa, sa, ib, bufb, sb, agg_sh):
    cid = lax.axis_index("c")
    sid = lax.axis_index("s")
    w = sid * NC + cid
    slots = ((ia, bufa, sa), (ib, bufb, sb))

    zeros = jnp.zeros((L,), jnp.float32)

    def zrow(r, carry):
        for j in range(D_HID // L):
            bufa[r, pl.ds(j * L, L)] = zeros
        return carry

    lax.fori_loop(0, ZROWS, zrow, 0)
    for k in range(ROWS_PER_TILE // ZROWS):
        pltpu.sync_copy(bufa, agg_sh.at[pl.ds(sid * ROWS_PER_TILE + k * ZROWS, ZROWS)])
    plsc.subcore_barrier()

    def start(i, slot):
        idxv, buf, sem = slot
        c = i * NW + w

        @pl.when(c < NCH)
        def _():
            base = c * CHUNK
            pltpu.sync_copy(dst_hbm.at[pl.ds(base, CHUNK)], idxv)
            pltpu.async_copy(hen_hbm.at[pl.ds(base, CHUNK)], buf, sem)

    def finish(i, slot):
        idxv, buf, sem = slot
        c = i * NW + w

        @pl.when(c < NCH)
        def _():
            base = c * CHUNK
            pltpu.make_async_copy(hen_hbm.at[pl.ds(base, CHUNK)], buf, sem).wait()
            pltpu.sync_copy(buf, agg_sh.at[idxv], add=True)

    for b in range(2):
        start(b, slots[b])

    def pair(p, carry):
        for b in range(2):
            i = p * 2 + b
            finish(i, slots[b])
            start(i + 2, slots[b])
        return carry

    lax.fori_loop(0, (ITERS + 1) // 2, pair, 0)
    plsc.subcore_barrier()

    for k in range(ROWS_PER_TILE // ZROWS):
        r0 = sid * ROWS_PER_TILE + k * ZROWS
        pltpu.sync_copy(agg_sh.at[pl.ds(r0, ZROWS)], agg_hbm.at[cid, pl.ds(r0, ZROWS)])


def _scatter(dst, he_new):
    slot = [
        pltpu.VMEM((CHUNK,), jnp.int32),
        pltpu.VMEM((CHUNK, D_HID), jnp.float32),
        pltpu.SemaphoreType.DMA,
    ]
    fn = pl.kernel(
        _scatter_body,
        out_type=jax.ShapeDtypeStruct((NC, N_PAD, D_HID), jnp.float32),
        mesh=plsc.VectorSubcoreMesh(
            core_axis_name="c", subcore_axis_name="s",
            num_cores=NC, num_subcores=NS),
        scratch_types=slot + slot + [pltpu.VMEM_SHARED((N_PAD, D_HID), jnp.float32)],
    )
    return fn(dst, he_new)


def _node_body(hn_ref, a0_ref, a1_ref, wn1a_ref, wn1b_ref, bn1_ref,
               wn2_ref, bn2_ref, wp_ref, bp_ref, out_ref):
    agg = a0_ref[...] + a1_ref[...]
    u = (jnp.dot(hn_ref[...], wn1a_ref[...], preferred_element_type=jnp.float32)
         + jnp.dot(agg, wn1b_ref[...], preferred_element_type=jnp.float32)
         + bn1_ref[...])
    u = u * jax.nn.sigmoid(u)
    v = jnp.dot(u, wn2_ref[...], preferred_element_type=jnp.float32) + bn2_ref[...]
    out_ref[...] = jnp.dot(v, wp_ref[...], preferred_element_type=jnp.float32) + bp_ref[...]


def _node_mlp(hn, a0, a1, wn1a, wn1b, bn1, wn2, bn2, wp, bp):
    blk = 1000
    return pl.pallas_call(
        _node_body,
        grid=(N // blk,),
        in_specs=[
            pl.BlockSpec((blk, D_LAT), lambda i: (i, 0)),
            pl.BlockSpec((blk, D_HID), lambda i: (i, 0)),
            pl.BlockSpec((blk, D_HID), lambda i: (i, 0)),
            pl.BlockSpec((D_LAT, D_HID), lambda i: (0, 0)),
            pl.BlockSpec((D_HID, D_HID), lambda i: (0, 0)),
            pl.BlockSpec((1, D_HID), lambda i: (0, 0)),
            pl.BlockSpec((D_HID, D_HID), lambda i: (0, 0)),
            pl.BlockSpec((1, D_HID), lambda i: (0, 0)),
            pl.BlockSpec((D_HID, D_OUT), lambda i: (0, 0)),
            pl.BlockSpec((1, D_OUT), lambda i: (0, 0)),
        ],
        out_specs=pl.BlockSpec((blk, D_OUT), lambda i: (i, 0)),
        out_shape=jax.ShapeDtypeStruct((N, D_OUT), jnp.float32),
    )(hn, a0, a1, wn1a, wn1b, bn1, wn2, bn2, wp, bp)


def kernel(hn, he, edge_index, We1, be1, We2, be2, Wn1, bn1, Wn2, bn2, Wp, bp):
    src = edge_index[0]
    dst = edge_index[1]
    we1a = We1[:D_LAT]
    we1b = We1[D_LAT:2 * D_LAT]
    we1c = We1[2 * D_LAT:]

    p1, p2 = _precompute(hn, we1a, we1b, be1.reshape(1, -1))
    p1p = jax.lax.bitcast_convert_type(p1.reshape(N, D_HID // 2, 2), jnp.float32)
    p2p = jax.lax.bitcast_convert_type(p2.reshape(N, D_HID // 2, 2), jnp.float32)
    tpk = jnp.concatenate([p1p, p2p], axis=1)
    tpk = jnp.pad(tpk, ((0, N_PAD - N), (0, 0)))
    g = _gather(src, dst, tpk)
    perm = jnp.concatenate([jnp.arange(0, D_HID, 2), jnp.arange(1, D_HID, 2)])
    he_new = _edge_mlp(g[0], g[1], he, we1c[:, perm], We2[perm, :],
                       be2.reshape(1, -1))
    aggs = _scatter(dst, he_new)
    hn_out = _node_mlp(hn, aggs[0], aggs[1],
                       Wn1[:D_LAT], Wn1[D_LAT:], bn1.reshape(1, -1),
                       Wn2, bn2.reshape(1, -1), Wp, bp.reshape(1, -1))
    return hn_out, he_new


# speedup vs baseline: 1.3789x; 1.3789x over previous
import jax
import jax.numpy as jnp
from jax import lax
from jax.experimental import pallas as pl
from jax.experimental.pallas import tpu as pltpu
from jax.experimental.pallas import tpu_sc as plsc

N = 10000
E = 320000
D_LAT = 128
D_EDGE = 16
D_HID = 128
D_OUT = 64

NC = 2
NS = 16
NW = NC * NS
L = 16

CHUNK = 128
HCH = CHUNK // 2
NCH = E // CHUNK
ITERS = (NCH + NW - 1) // NW
ITERS2 = (NCH + NS - 1) // NS
N_PAD = 10240
NHALF = N_PAD // 2
TROWS_T = NHALF // NS
ROWS_PER_TILE = N_PAD // NS
ZROWS = CHUNK


def _pre_body(hn_ref, wa_ref, wb_ref, b1_ref, p1_ref, p2_ref):
    h = hn_ref[...]
    p1 = jnp.dot(h, wa_ref[...], preferred_element_type=jnp.float32) + b1_ref[...]
    p2 = jnp.dot(h, wb_ref[...], preferred_element_type=jnp.float32)
    p1_ref[...] = p1.astype(jnp.bfloat16)
    p2_ref[...] = p2.astype(jnp.bfloat16)


def _precompute(hn, we1a, we1b, be1):
    blk = 1000
    return pl.pallas_call(
        _pre_body,
        grid=(N // blk,),
        in_specs=[
            pl.BlockSpec((blk, D_LAT), lambda i: (i, 0)),
            pl.BlockSpec((D_LAT, D_HID), lambda i: (0, 0)),
            pl.BlockSpec((D_LAT, D_HID), lambda i: (0, 0)),
            pl.BlockSpec((1, D_HID), lambda i: (0, 0)),
        ],
        out_specs=[
            pl.BlockSpec((blk, D_HID), lambda i: (i, 0)),
            pl.BlockSpec((blk, D_HID), lambda i: (i, 0)),
        ],
        out_shape=[
            jax.ShapeDtypeStruct((N, D_HID), jnp.bfloat16),
            jax.ShapeDtypeStruct((N, D_HID), jnp.bfloat16),
        ],
    )(hn, we1a, we1b, be1)


def _gather_body(src_hbm, dst_hbm, t_hbm, g_hbm,
                 i1a, i2a, b1a, b2a, bga, sga,
                 i1b, i2b, b1b, b2b, bgb, sgb):
    cid = lax.axis_index("c")
    sid = lax.axis_index("s")
    w = sid * NC + cid

    slots = ((i1a, i2a, b1a, b2a, bga, sga), (i1b, i2b, b1b, b2b, bgb, sgb))

    def start(i, slot):
        i1, i2, b1, b2, bg, sg = slot
        c = i * NW + w

        @pl.when(c < NCH)
        def _():
            base = c * CHUNK
            pltpu.sync_copy(src_hbm.at[pl.ds(base, CHUNK)], i1)
            pltpu.sync_copy(dst_hbm.at[pl.ds(base, CHUNK)], i2)
            pltpu.async_copy(t_hbm.at[i1], b1, sg)
            pltpu.async_copy(t_hbm.at[i2], b2, sg)

    def finish(i, slot):
        i1, i2, b1, b2, bg, sg = slot
        c = i * NW + w

        @pl.when(c < NCH)
        def _():
            pltpu.make_async_copy(t_hbm.at[i1], b1, sg).wait()
            pltpu.make_async_copy(t_hbm.at[i2], b2, sg).wait()

            M = jnp.uint32(0xFFFF0000)
            R = jnp.uint32(0x8000)

            def packadd(w1, w2):
                u1 = jax.lax.bitcast_convert_type(w1, jnp.uint32)
                u2 = jax.lax.bitcast_convert_type(w2, jnp.uint32)
                lo = (jax.lax.bitcast_convert_type(u1 << 16, jnp.float32)
                      + jax.lax.bitcast_convert_type(u2 << 16, jnp.float32))
                hi = (jax.lax.bitcast_convert_type(u1 & M, jnp.float32)
                      + jax.lax.bitcast_convert_type(u2 & M, jnp.float32))
                ul = (jax.lax.bitcast_convert_type(lo, jnp.uint32) + R) >> 16
                uh = (jax.lax.bitcast_convert_type(hi, jnp.uint32) + R) & M
                return jax.lax.bitcast_convert_type(ul | uh, jnp.float32)

            def pairrow(t, rcarry):
                rb = t + HCH
                for j in range(64 // L):
                    cs = pl.ds(j * L, L)
                    cs2 = pl.ds(64 + j * L, L)
                    bg[t, cs] = packadd(b1[t, cs], b2[t, cs2])
                    bg[t, cs2] = packadd(b1[rb, cs], b2[rb, cs2])
                return rcarry

            lax.fori_loop(0, HCH, pairrow, 0)
            pltpu.sync_copy(bg, g_hbm.at[pl.ds(c * HCH, HCH)])

    for b in range(2):
        start(b, slots[b])

    def pair(p, carry):
        for b in range(2):
            i = p * 2 + b
            finish(i, slots[b])
            start(i + 2, slots[b])
        return carry

    lax.fori_loop(0, (ITERS + 1) // 2, pair, 0)


def _gather(src, dst, tpk):
    slot = [
        pltpu.VMEM((CHUNK,), jnp.int32),
        pltpu.VMEM((CHUNK,), jnp.int32),
        pltpu.VMEM((CHUNK, D_HID), jnp.float32),
        pltpu.VMEM((CHUNK, D_HID), jnp.float32),
        pltpu.VMEM((HCH, D_HID), jnp.float32),
        pltpu.SemaphoreType.DMA,
    ]
    fn = pl.kernel(
        _gather_body,
        out_type=jax.ShapeDtypeStruct((E // 2, D_HID), jnp.float32),
        mesh=plsc.VectorSubcoreMesh(
            core_axis_name="c", subcore_axis_name="s",
            num_cores=NC, num_subcores=NS),
        scratch_types=slot + slot,
    )
    return fn(src, dst, tpk)


KPB = 10
BLK_E = KPB * CHUNK
BLK_G = KPB * HCH


def _unpack(g):
    u = jax.lax.bitcast_convert_type(g, jnp.uint32)
    lo = jax.lax.bitcast_convert_type(u << 16, jnp.float32)
    hi = jax.lax.bitcast_convert_type(u & jnp.uint32(0xFFFF0000), jnp.float32)
    return lo, hi


def _edge_body(g_ref, he_ref, wc_ref, w2_ref, b2_ref, out_ref):
    lo, hi = _unpack(g_ref[...])
    xa = jnp.concatenate([lo[:, :64], hi[:, :64]], axis=1)
    xb = jnp.concatenate([lo[:, 64:], hi[:, 64:]], axis=1)
    x = jnp.concatenate([xa, xb], axis=0)
    he_blk = he_ref[...]
    hea = jnp.concatenate(
        [he_blk[CHUNK * k:CHUNK * k + HCH] for k in range(KPB)], axis=0)
    heb = jnp.concatenate(
        [he_blk[CHUNK * k + HCH:CHUNK * (k + 1)] for k in range(KPB)], axis=0)
    he_sel = jnp.concatenate([hea, heb], axis=0)
    x = x + jnp.dot(he_sel, wc_ref[...], preferred_element_type=jnp.float32)
    s = x * jax.nn.sigmoid(x)
    o = jnp.dot(s, w2_ref[...], preferred_element_type=jnp.float32) + b2_ref[...]
    for k in range(KPB):
        out_ref[pl.ds(CHUNK * k, HCH), :] = o[HCH * k:HCH * (k + 1)]
        out_ref[pl.ds(CHUNK * k + HCH, HCH), :] = o[BLK_G + HCH * k:BLK_G + HCH * (k + 1)]


def _edge_mlp(g, he, we1c, we2, be2):
    return pl.pallas_call(
        _edge_body,
        grid=(E // BLK_E,),
        in_specs=[
            pl.BlockSpec((BLK_G, D_HID), lambda i: (i, 0)),
            pl.BlockSpec((BLK_E, D_EDGE), lambda i: (i, 0)),
            pl.BlockSpec((D_EDGE, D_HID), lambda i: (0, 0)),
            pl.BlockSpec((D_HID, D_HID), lambda i: (0, 0)),
            pl.BlockSpec((1, D_HID), lambda i: (0, 0)),
        ],
        out_specs=pl.BlockSpec((BLK_E, D_HID), lambda i: (i, 0)),
        out_shape=jax.ShapeDtypeStruct((E, D_HID), jnp.float32),
    )(g, he, we1c, we2, be2)


def _scatter_body(dst_hbm, hen_hbm, agg_hbm,
                  ia, bufa, sa, ib, bufb, sb, agg_sh):
    cid = lax.axis_index("c")
    sid = lax.axis_index("s")
    w = sid * NC + cid
    slots = ((ia, bufa, sa), (ib, bufb, sb))

    zeros = jnp.zeros((L,), jnp.float32)

    def zrow(r, carry):
        for j in range(D_HID // L):
            bufa[r, pl.ds(j * L, L)] = zeros
        return carry

    lax.fori_loop(0, ZROWS, zrow, 0)
    for k in range(ROWS_PER_TILE // ZROWS):
        pltpu.sync_copy(bufa, agg_sh.at[pl.ds(sid * ROWS_PER_TILE + k * ZROWS, ZROWS)])
    plsc.subcore_barrier()

    def start(i, slot):
        idxv, buf, sem = slot
        c = i * NW + w

        @pl.when(c < NCH)
        def _():
            base = c * CHUNK
            pltpu.sync_copy(dst_hbm.at[pl.ds(base, CHUNK)], idxv)
            pltpu.async_copy(hen_hbm.at[pl.ds(base, CHUNK)], buf, sem)

    def finish(i, slot):
        idxv, buf, sem = slot
        c = i * NW + w

        @pl.when(c < NCH)
        def _():
            base = c * CHUNK
            pltpu.make_async_copy(hen_hbm.at[pl.ds(base, CHUNK)], buf, sem).wait()
            pltpu.sync_copy(buf, agg_sh.at[idxv], add=True)

    for b in range(2):
        start(b, slots[b])

    def pair(p, carry):
        for b in range(2):
            i = p * 2 + b
            finish(i, slots[b])
            start(i + 2, slots[b])
        return carry

    lax.fori_loop(0, (ITERS + 1) // 2, pair, 0)
    plsc.subcore_barrier()

    for k in range(ROWS_PER_TILE // ZROWS):
        r0 = sid * ROWS_PER_TILE + k * ZROWS
        pltpu.sync_copy(agg_sh.at[pl.ds(r0, ZROWS)], agg_hbm.at[cid, pl.ds(r0, ZROWS)])


def _scatter(dst, he_new):
    slot = [
        pltpu.VMEM((CHUNK,), jnp.int32),
        pltpu.VMEM((CHUNK, D_HID), jnp.float32),
        pltpu.SemaphoreType.DMA,
    ]
    fn = pl.kernel(
        _scatter_body,
        out_type=jax.ShapeDtypeStruct((NC, N_PAD, D_HID), jnp.float32),
        mesh=plsc.VectorSubcoreMesh(
            core_axis_name="c", subcore_axis_name="s",
            num_cores=NC, num_subcores=NS),
        scratch_types=slot + slot + [pltpu.VMEM_SHARED((N_PAD, D_HID), jnp.float32)],
    )
    return fn(dst, he_new)


def _node_body(hn_ref, a0_ref, a1_ref, wn1a_ref, wn1b_ref, bn1_ref,
               wn2_ref, bn2_ref, wp_ref, bp_ref, out_ref):
    agg = a0_ref[...] + a1_ref[...]
    u = (jnp.dot(hn_ref[...], wn1a_ref[...], preferred_element_type=jnp.float32)
         + jnp.dot(agg, wn1b_ref[...], preferred_element_type=jnp.float32)
         + bn1_ref[...])
    u = u * jax.nn.sigmoid(u)
    v = jnp.dot(u, wn2_ref[...], preferred_element_type=jnp.float32) + bn2_ref[...]
    out_ref[...] = jnp.dot(v, wp_ref[...], preferred_element_type=jnp.float32) + bp_ref[...]


def _node_mlp(hn, a0, a1, wn1a, wn1b, bn1, wn2, bn2, wp, bp):
    blk = 1000
    return pl.pallas_call(
        _node_body,
        grid=(N // blk,),
        in_specs=[
            pl.BlockSpec((blk, D_LAT), lambda i: (i, 0)),
            pl.BlockSpec((blk, D_HID), lambda i: (i, 0)),
            pl.BlockSpec((blk, D_HID), lambda i: (i, 0)),
            pl.BlockSpec((D_LAT, D_HID), lambda i: (0, 0)),
            pl.BlockSpec((D_HID, D_HID), lambda i: (0, 0)),
            pl.BlockSpec((1, D_HID), lambda i: (0, 0)),
            pl.BlockSpec((D_HID, D_HID), lambda i: (0, 0)),
            pl.BlockSpec((1, D_HID), lambda i: (0, 0)),
            pl.BlockSpec((D_HID, D_OUT), lambda i: (0, 0)),
            pl.BlockSpec((1, D_OUT), lambda i: (0, 0)),
        ],
        out_specs=pl.BlockSpec((blk, D_OUT), lambda i: (i, 0)),
        out_shape=jax.ShapeDtypeStruct((N, D_OUT), jnp.float32),
    )(hn, a0, a1, wn1a, wn1b, bn1, wn2, bn2, wp, bp)


def kernel(hn, he, edge_index, We1, be1, We2, be2, Wn1, bn1, Wn2, bn2, Wp, bp):
    src = edge_index[0]
    dst = edge_index[1]
    we1a = We1[:D_LAT]
    we1b = We1[D_LAT:2 * D_LAT]
    we1c = We1[2 * D_LAT:]

    p1, p2 = _precompute(hn, we1a, we1b, be1.reshape(1, -1))
    p1p = jax.lax.bitcast_convert_type(p1.reshape(N, D_HID // 2, 2), jnp.float32)
    p2p = jax.lax.bitcast_convert_type(p2.reshape(N, D_HID // 2, 2), jnp.float32)
    tpk = jnp.concatenate([p1p, p2p], axis=1)
    g = _gather(src, dst, tpk)
    perm = jnp.concatenate([jnp.arange(0, D_HID, 2), jnp.arange(1, D_HID, 2)])
    he_new = _edge_mlp(g, he, we1c[:, perm], We2[perm, :], be2.reshape(1, -1))
    aggs = _scatter(dst, he_new)
    hn_out = _node_mlp(hn, aggs[0], aggs[1],
                       Wn1[:D_LAT], Wn1[D_LAT:], bn1.reshape(1, -1),
                       Wn2, bn2.reshape(1, -1), Wp, bp.reshape(1, -1))
    return hn_out, he_new


# speedup vs baseline: 1.5014x; 1.0889x over previous
import jax
import jax.numpy as jnp
from jax import lax
from jax.experimental import pallas as pl
from jax.experimental.pallas import tpu as pltpu
from jax.experimental.pallas import tpu_sc as plsc

N = 10000
E = 320000
D_LAT = 128
D_EDGE = 16
D_HID = 128
D_OUT = 64

NC = 2
NS = 16
NW = NC * NS
L = 16

CHUNK = 128
HCH = CHUNK // 2
NCH = E // CHUNK
ITERS = (NCH + NW - 1) // NW
ITERS2 = (NCH + NS - 1) // NS
N_PAD = 10240
NHALF = N_PAD // 2
TROWS_T = NHALF // NS
ROWS_PER_TILE = N_PAD // NS
ZROWS = CHUNK


def _pre_body(hn_ref, wa_ref, wb_ref, b1_ref, p1_ref, p2_ref):
    h = hn_ref[...]
    p1 = jnp.dot(h, wa_ref[...], preferred_element_type=jnp.float32) + b1_ref[...]
    p2 = jnp.dot(h, wb_ref[...], preferred_element_type=jnp.float32)
    p1_ref[...] = p1.astype(jnp.bfloat16)
    p2_ref[...] = p2.astype(jnp.bfloat16)


def _precompute(hn, we1a, we1b, be1):
    blk = 1000
    return pl.pallas_call(
        _pre_body,
        grid=(N // blk,),
        in_specs=[
            pl.BlockSpec((blk, D_LAT), lambda i: (i, 0)),
            pl.BlockSpec((D_LAT, D_HID), lambda i: (0, 0)),
            pl.BlockSpec((D_LAT, D_HID), lambda i: (0, 0)),
            pl.BlockSpec((1, D_HID), lambda i: (0, 0)),
        ],
        out_specs=[
            pl.BlockSpec((blk, D_HID), lambda i: (i, 0)),
            pl.BlockSpec((blk, D_HID), lambda i: (i, 0)),
        ],
        out_shape=[
            jax.ShapeDtypeStruct((N, D_HID), jnp.bfloat16),
            jax.ShapeDtypeStruct((N, D_HID), jnp.bfloat16),
        ],
    )(hn, we1a, we1b, be1)


def _make_gather_body(c0, c1):
    iters = (c1 - c0 + NW - 1) // NW

    def _gather_body(src_hbm, dst_hbm, t_hbm, g_hbm,
                     i1a, i2a, b1a, b2a, bga, sga,
                     i1b, i2b, b1b, b2b, bgb, sgb):
        cid = lax.axis_index("c")
        sid = lax.axis_index("s")
        w = sid * NC + cid

        slots = ((i1a, i2a, b1a, b2a, bga, sga), (i1b, i2b, b1b, b2b, bgb, sgb))

        def start(i, slot):
            i1, i2, b1, b2, bg, sg = slot
            c = c0 + i * NW + w

            @pl.when(c < c1)
            def _():
                base = c * CHUNK
                pltpu.sync_copy(src_hbm.at[pl.ds(base, CHUNK)], i1)
                pltpu.sync_copy(dst_hbm.at[pl.ds(base, CHUNK)], i2)
                pltpu.async_copy(t_hbm.at[i1], b1, sg)
                pltpu.async_copy(t_hbm.at[i2], b2, sg)

        def finish(i, slot):
            i1, i2, b1, b2, bg, sg = slot
            c = c0 + i * NW + w

            @pl.when(c < c1)
            def _():
                pltpu.make_async_copy(t_hbm.at[i1], b1, sg).wait()
                pltpu.make_async_copy(t_hbm.at[i2], b2, sg).wait()

                M = jnp.uint32(0xFFFF0000)
                R = jnp.uint32(0x8000)

                def packadd(w1, w2):
                    u1 = jax.lax.bitcast_convert_type(w1, jnp.uint32)
                    u2 = jax.lax.bitcast_convert_type(w2, jnp.uint32)
                    lo = (jax.lax.bitcast_convert_type(u1 << 16, jnp.float32)
                          + jax.lax.bitcast_convert_type(u2 << 16, jnp.float32))
                    hi = (jax.lax.bitcast_convert_type(u1 & M, jnp.float32)
                          + jax.lax.bitcast_convert_type(u2 & M, jnp.float32))
                    ul = (jax.lax.bitcast_convert_type(lo, jnp.uint32) + R) >> 16
                    uh = (jax.lax.bitcast_convert_type(hi, jnp.uint32) + R) & M
                    return jax.lax.bitcast_convert_type(ul | uh, jnp.float32)

                def pairrow(t, rcarry):
                    rb = t + HCH
                    for j in range(64 // L):
                        cs = pl.ds(j * L, L)
                        cs2 = pl.ds(64 + j * L, L)
                        bg[t, cs] = packadd(b1[t, cs], b2[t, cs2])
                        bg[t, cs2] = packadd(b1[rb, cs], b2[rb, cs2])
                    return rcarry

                lax.fori_loop(0, HCH, pairrow, 0)
                pltpu.sync_copy(bg, g_hbm.at[pl.ds((c - c0) * HCH, HCH)])

        for b in range(2):
            start(b, slots[b])

        def pair(p, carry):
            for b in range(2):
                i = p * 2 + b
                finish(i, slots[b])
                start(i + 2, slots[b])
            return carry

        lax.fori_loop(0, (iters + 1) // 2, pair, 0)

    return _gather_body


def _gather(src, dst, tpk, c0, c1):
    slot = [
        pltpu.VMEM((CHUNK,), jnp.int32),
        pltpu.VMEM((CHUNK,), jnp.int32),
        pltpu.VMEM((CHUNK, D_HID), jnp.float32),
        pltpu.VMEM((CHUNK, D_HID), jnp.float32),
        pltpu.VMEM((HCH, D_HID), jnp.float32),
        pltpu.SemaphoreType.DMA,
    ]
    fn = pl.kernel(
        _make_gather_body(c0, c1),
        out_type=jax.ShapeDtypeStruct(((c1 - c0) * HCH, D_HID), jnp.float32),
        mesh=plsc.VectorSubcoreMesh(
            core_axis_name="c", subcore_axis_name="s",
            num_cores=NC, num_subcores=NS),
        scratch_types=slot + slot,
    )
    return fn(src, dst, tpk)


KPB = 10
BLK_E = KPB * CHUNK
BLK_G = KPB * HCH


def _unpack(g):
    u = jax.lax.bitcast_convert_type(g, jnp.uint32)
    lo = jax.lax.bitcast_convert_type(u << 16, jnp.float32)
    hi = jax.lax.bitcast_convert_type(u & jnp.uint32(0xFFFF0000), jnp.float32)
    return lo, hi


def _edge_body(g_ref, he_ref, wc_ref, w2_ref, b2_ref, out_ref):
    lo, hi = _unpack(g_ref[...])
    xa = jnp.concatenate([lo[:, :64], hi[:, :64]], axis=1)
    xb = jnp.concatenate([lo[:, 64:], hi[:, 64:]], axis=1)
    x = jnp.concatenate([xa, xb], axis=0)
    he_blk = he_ref[...]
    hea = jnp.concatenate(
        [he_blk[CHUNK * k:CHUNK * k + HCH] for k in range(KPB)], axis=0)
    heb = jnp.concatenate(
        [he_blk[CHUNK * k + HCH:CHUNK * (k + 1)] for k in range(KPB)], axis=0)
    he_sel = jnp.concatenate([hea, heb], axis=0)
    x = x + jnp.dot(he_sel, wc_ref[...], preferred_element_type=jnp.float32)
    s = x * jax.nn.sigmoid(x)
    o = jnp.dot(s, w2_ref[...], preferred_element_type=jnp.float32) + b2_ref[...]
    for k in range(KPB):
        out_ref[pl.ds(CHUNK * k, HCH), :] = o[HCH * k:HCH * (k + 1)]
        out_ref[pl.ds(CHUNK * k + HCH, HCH), :] = o[BLK_G + HCH * k:BLK_G + HCH * (k + 1)]


def _edge_body_p1(g_ref, he_ref, wc_ref, w2_ref, b2_ref, prev_ref, out_ref):
    _edge_body(g_ref, he_ref, wc_ref, w2_ref, b2_ref, out_ref)


def _edge_mlp(g, he, we1c, we2, be2, part, prev=None):
    nb = E // BLK_E // 2
    off = part * nb
    in_specs = [
        pl.BlockSpec((BLK_G, D_HID), lambda i: (i, 0)),
        pl.BlockSpec((BLK_E, D_EDGE), lambda i: (i + off, 0)),
        pl.BlockSpec((D_EDGE, D_HID), lambda i: (0, 0)),
        pl.BlockSpec((D_HID, D_HID), lambda i: (0, 0)),
        pl.BlockSpec((1, D_HID), lambda i: (0, 0)),
    ]
    args = (g, he, we1c, we2, be2)
    body = _edge_body
    aliases = {}
    if part == 1:
        in_specs.append(pl.BlockSpec(memory_space=pltpu.MemorySpace.HBM))
        args = args + (prev,)
        body = _edge_body_p1
        aliases = {5: 0}
    return pl.pallas_call(
        body,
        grid=(nb,),
        in_specs=in_specs,
        out_specs=pl.BlockSpec((BLK_E, D_HID), lambda i: (i + off, 0)),
        out_shape=jax.ShapeDtypeStruct((E, D_HID), jnp.float32),
        input_output_aliases=aliases,
    )(*args)


def _scatter_body(dst_hbm, hen_hbm, agg_hbm,
                  ia, bufa, sa, ib, bufb, sb, agg_sh):
    cid = lax.axis_index("c")
    sid = lax.axis_index("s")
    w = sid * NC + cid
    slots = ((ia, bufa, sa), (ib, bufb, sb))

    zeros = jnp.zeros((L,), jnp.float32)

    def zrow(r, carry):
        for j in range(D_HID // L):
            bufa[r, pl.ds(j * L, L)] = zeros
        return carry

    lax.fori_loop(0, ZROWS, zrow, 0)
    for k in range(ROWS_PER_TILE // ZROWS):
        pltpu.sync_copy(bufa, agg_sh.at[pl.ds(sid * ROWS_PER_TILE + k * ZROWS, ZROWS)])
    plsc.subcore_barrier()

    def start(i, slot):
        idxv, buf, sem = slot
        c = i * NW + w

        @pl.when(c < NCH)
        def _():
            base = c * CHUNK
            pltpu.sync_copy(dst_hbm.at[pl.ds(base, CHUNK)], idxv)
            pltpu.async_copy(hen_hbm.at[pl.ds(base, CHUNK)], buf, sem)

    def finish(i, slot):
        idxv, buf, sem = slot
        c = i * NW + w

        @pl.when(c < NCH)
        def _():
            base = c * CHUNK
            pltpu.make_async_copy(hen_hbm.at[pl.ds(base, CHUNK)], buf, sem).wait()
            pltpu.sync_copy(buf, agg_sh.at[idxv], add=True)

    for b in range(2):
        start(b, slots[b])

    def pair(p, carry):
        for b in range(2):
            i = p * 2 + b
            finish(i, slots[b])
            start(i + 2, slots[b])
        return carry

    lax.fori_loop(0, (ITERS + 1) // 2, pair, 0)
    plsc.subcore_barrier()

    for k in range(ROWS_PER_TILE // ZROWS):
        r0 = sid * ROWS_PER_TILE + k * ZROWS
        pltpu.sync_copy(agg_sh.at[pl.ds(r0, ZROWS)], agg_hbm.at[cid, pl.ds(r0, ZROWS)])


def _scatter(dst, he_new):
    slot = [
        pltpu.VMEM((CHUNK,), jnp.int32),
        pltpu.VMEM((CHUNK, D_HID), jnp.float32),
        pltpu.SemaphoreType.DMA,
    ]
    fn = pl.kernel(
        _scatter_body,
        out_type=jax.ShapeDtypeStruct((NC, N_PAD, D_HID), jnp.float32),
        mesh=plsc.VectorSubcoreMesh(
            core_axis_name="c", subcore_axis_name="s",
            num_cores=NC, num_subcores=NS),
        scratch_types=slot + slot + [pltpu.VMEM_SHARED((N_PAD, D_HID), jnp.float32)],
    )
    return fn(dst, he_new)


def _node_body(hn_ref, a0_ref, a1_ref, wn1a_ref, wn1b_ref, bn1_ref,
               wn2_ref, bn2_ref, wp_ref, bp_ref, out_ref):
    agg = a0_ref[...] + a1_ref[...]
    u = (jnp.dot(hn_ref[...], wn1a_ref[...], preferred_element_type=jnp.float32)
         + jnp.dot(agg, wn1b_ref[...], preferred_element_type=jnp.float32)
         + bn1_ref[...])
    u = u * jax.nn.sigmoid(u)
    v = jnp.dot(u, wn2_ref[...], preferred_element_type=jnp.float32) + bn2_ref[...]
    out_ref[...] = jnp.dot(v, wp_ref[...], preferred_element_type=jnp.float32) + bp_ref[...]


def _node_mlp(hn, a0, a1, wn1a, wn1b, bn1, wn2, bn2, wp, bp):
    blk = 1000
    return pl.pallas_call(
        _node_body,
        grid=(N // blk,),
        in_specs=[
            pl.BlockSpec((blk, D_LAT), lambda i: (i, 0)),
            pl.BlockSpec((blk, D_HID), lambda i: (i, 0)),
            pl.BlockSpec((blk, D_HID), lambda i: (i, 0)),
            pl.BlockSpec((D_LAT, D_HID), lambda i: (0, 0)),
            pl.BlockSpec((D_HID, D_HID), lambda i: (0, 0)),
            pl.BlockSpec((1, D_HID), lambda i: (0, 0)),
            pl.BlockSpec((D_HID, D_HID), lambda i: (0, 0)),
            pl.BlockSpec((1, D_HID), lambda i: (0, 0)),
            pl.BlockSpec((D_HID, D_OUT), lambda i: (0, 0)),
            pl.BlockSpec((1, D_OUT), lambda i: (0, 0)),
        ],
        out_specs=pl.BlockSpec((blk, D_OUT), lambda i: (i, 0)),
        out_shape=jax.ShapeDtypeStruct((N, D_OUT), jnp.float32),
    )(hn, a0, a1, wn1a, wn1b, bn1, wn2, bn2, wp, bp)


def kernel(hn, he, edge_index, We1, be1, We2, be2, Wn1, bn1, Wn2, bn2, Wp, bp):
    src = edge_index[0]
    dst = edge_index[1]
    we1a = We1[:D_LAT]
    we1b = We1[D_LAT:2 * D_LAT]
    we1c = We1[2 * D_LAT:]

    p1, p2 = _precompute(hn, we1a, we1b, be1.reshape(1, -1))
    p1p = jax.lax.bitcast_convert_type(p1.reshape(N, D_HID // 2, 2), jnp.float32)
    p2p = jax.lax.bitcast_convert_type(p2.reshape(N, D_HID // 2, 2), jnp.float32)
    tpk = jnp.concatenate([p1p, p2p], axis=1)
    g1 = _gather(src, dst, tpk, 0, NCH // 2)
    g2 = _gather(src, dst, tpk, NCH // 2, NCH)
    perm = jnp.concatenate([jnp.arange(0, D_HID, 2), jnp.arange(1, D_HID, 2)])
    wc_p = we1c[:, perm]
    w2_p = We2[perm, :]
    b2r = be2.reshape(1, -1)
    h1 = _edge_mlp(g1, he, wc_p, w2_p, b2r, 0)
    he_new = _edge_mlp(g2, he, wc_p, w2_p, b2r, 1, prev=h1)
    aggs = _scatter(dst, he_new)
    hn_out = _node_mlp(hn, aggs[0], aggs[1],
                       Wn1[:D_LAT], Wn1[D_LAT:], bn1.reshape(1, -1),
                       Wn2, bn2.reshape(1, -1), Wp, bp.reshape(1, -1))
    return hn_out, he_new


# speedup vs baseline: 1.6631x; 1.1077x over previous
import jax
import jax.numpy as jnp
from jax import lax
from jax.experimental import pallas as pl
from jax.experimental.pallas import tpu as pltpu
from jax.experimental.pallas import tpu_sc as plsc

N = 10000
E = 320000
D_LAT = 128
D_EDGE = 16
D_HID = 128
D_OUT = 64

NC = 2
NS = 16
NW = NC * NS
L = 16

CHUNK = 128
HCH = CHUNK // 2
NCH = E // CHUNK
ITERS = (NCH + NW - 1) // NW
ITERS2 = (NCH + NS - 1) // NS
N_PAD = 10240
NHALF = N_PAD // 2
TROWS_T = NHALF // NS
ROWS_PER_TILE = N_PAD // NS
ZROWS = CHUNK


def _pre_body(hn_ref, wa_ref, wb_ref, b1_ref, t_ref):
    h = hn_ref[...]
    p1 = jnp.dot(h, wa_ref[...], preferred_element_type=jnp.float32) + b1_ref[...]
    p2 = jnp.dot(h, wb_ref[...], preferred_element_type=jnp.float32)
    u1 = jax.lax.bitcast_convert_type(p1, jnp.uint32)
    u2 = jax.lax.bitcast_convert_type(p2, jnp.uint32)
    r1 = (u1 + jnp.uint32(0x8000)) >> 16
    r2 = (u2 + jnp.uint32(0x8000)) & jnp.uint32(0xFFFF0000)
    t_ref[...] = jax.lax.bitcast_convert_type(r1 | r2, jnp.float32)


def _precompute(hn, we1a, we1b, be1):
    blk = 1000
    return pl.pallas_call(
        _pre_body,
        grid=(N // blk,),
        in_specs=[
            pl.BlockSpec((blk, D_LAT), lambda i: (i, 0)),
            pl.BlockSpec((D_LAT, D_HID), lambda i: (0, 0)),
            pl.BlockSpec((D_LAT, D_HID), lambda i: (0, 0)),
            pl.BlockSpec((1, D_HID), lambda i: (0, 0)),
        ],
        out_specs=pl.BlockSpec((blk, D_HID), lambda i: (i, 0)),
        out_shape=jax.ShapeDtypeStruct((N, D_HID), jnp.float32),
    )(hn, we1a, we1b, be1)


def _make_gather_body(c0, c1):
    iters = (c1 - c0 + NW - 1) // NW

    def _gather_body(src_hbm, dst_hbm, t_hbm, g_hbm,
                     i1a, i2a, b1a, b2a, bga, sga,
                     i1b, i2b, b1b, b2b, bgb, sgb):
        cid = lax.axis_index("c")
        sid = lax.axis_index("s")
        w = sid * NC + cid

        slots = ((i1a, i2a, b1a, b2a, bga, sga), (i1b, i2b, b1b, b2b, bgb, sgb))

        def start(i, slot):
            i1, i2, b1, b2, bg, sg = slot
            c = c0 + i * NW + w

            @pl.when(c < c1)
            def _():
                base = c * CHUNK
                pltpu.sync_copy(src_hbm.at[pl.ds(base, CHUNK)], i1)
                pltpu.sync_copy(dst_hbm.at[pl.ds(base, CHUNK)], i2)
                pltpu.async_copy(t_hbm.at[i1], b1, sg)
                pltpu.async_copy(t_hbm.at[i2], b2, sg)

        def finish(i, slot):
            i1, i2, b1, b2, bg, sg = slot
            c = c0 + i * NW + w

            @pl.when(c < c1)
            def _():
                pltpu.make_async_copy(t_hbm.at[i1], b1, sg).wait()
                pltpu.make_async_copy(t_hbm.at[i2], b2, sg).wait()

                M = jnp.uint32(0xFFFF0000)
                R = jnp.uint32(0x8000)

                def colsum(w1, w2):
                    u1 = jax.lax.bitcast_convert_type(w1, jnp.uint32)
                    u2 = jax.lax.bitcast_convert_type(w2, jnp.uint32)
                    return (jax.lax.bitcast_convert_type(u1 << 16, jnp.float32)
                            + jax.lax.bitcast_convert_type(u2 & M, jnp.float32))

                def pairrow(t, rcarry):
                    rb = t + HCH
                    for j in range(D_HID // L):
                        cs = pl.ds(j * L, L)
                        a = colsum(b1[t, cs], b2[t, cs])
                        b = colsum(b1[rb, cs], b2[rb, cs])
                        ua = (jax.lax.bitcast_convert_type(a, jnp.uint32) + R) >> 16
                        ub = (jax.lax.bitcast_convert_type(b, jnp.uint32) + R) & M
                        bg[t, cs] = jax.lax.bitcast_convert_type(ua | ub, jnp.float32)
                    return rcarry

                lax.fori_loop(0, HCH, pairrow, 0)
                pltpu.sync_copy(bg, g_hbm.at[pl.ds((c - c0) * HCH, HCH)])

        for b in range(2):
            start(b, slots[b])

        def pair(p, carry):
            for b in range(2):
                i = p * 2 + b
                finish(i, slots[b])
                start(i + 2, slots[b])
            return carry

        lax.fori_loop(0, (iters + 1) // 2, pair, 0)

    return _gather_body


def _gather(src, dst, tpk, c0, c1):
    slot = [
        pltpu.VMEM((CHUNK,), jnp.int32),
        pltpu.VMEM((CHUNK,), jnp.int32),
        pltpu.VMEM((CHUNK, D_HID), jnp.float32),
        pltpu.VMEM((CHUNK, D_HID), jnp.float32),
        pltpu.VMEM((HCH, D_HID), jnp.float32),
        pltpu.SemaphoreType.DMA,
    ]
    fn = pl.kernel(
        _make_gather_body(c0, c1),
        out_type=jax.ShapeDtypeStruct(((c1 - c0) * HCH, D_HID), jnp.float32),
        mesh=plsc.VectorSubcoreMesh(
            core_axis_name="c", subcore_axis_name="s",
            num_cores=NC, num_subcores=NS),
        scratch_types=slot + slot,
    )
    return fn(src, dst, tpk)


KPB = 10
BLK_E = KPB * CHUNK
BLK_G = KPB * HCH


def _unpack(g):
    u = jax.lax.bitcast_convert_type(g, jnp.uint32)
    lo = jax.lax.bitcast_convert_type(u << 16, jnp.float32)
    hi = jax.lax.bitcast_convert_type(u & jnp.uint32(0xFFFF0000), jnp.float32)
    return lo, hi


def _edge_body(g_ref, he_ref, wc_ref, w2_ref, b2_ref, out_ref):
    lo, hi = _unpack(g_ref[...])
    he_blk = he_ref[...]
    hea = jnp.concatenate(
        [he_blk[CHUNK * k:CHUNK * k + HCH] for k in range(KPB)], axis=0)
    heb = jnp.concatenate(
        [he_blk[CHUNK * k + HCH:CHUNK * (k + 1)] for k in range(KPB)], axis=0)
    xa = lo + jnp.dot(hea, wc_ref[...], preferred_element_type=jnp.float32)
    xb = hi + jnp.dot(heb, wc_ref[...], preferred_element_type=jnp.float32)
    sa = xa * jax.nn.sigmoid(xa)
    sb = xb * jax.nn.sigmoid(xb)
    oa = jnp.dot(sa, w2_ref[...], preferred_element_type=jnp.float32) + b2_ref[...]
    ob = jnp.dot(sb, w2_ref[...], preferred_element_type=jnp.float32) + b2_ref[...]
    for k in range(KPB):
        out_ref[pl.ds(CHUNK * k, HCH), :] = oa[HCH * k:HCH * (k + 1)]
        out_ref[pl.ds(CHUNK * k + HCH, HCH), :] = ob[HCH * k:HCH * (k + 1)]


def _edge_body_p1(g_ref, he_ref, wc_ref, w2_ref, b2_ref, prev_ref, out_ref):
    _edge_body(g_ref, he_ref, wc_ref, w2_ref, b2_ref, out_ref)


def _edge_mlp(g, he, we1c, we2, be2, part, prev=None):
    nb = E // BLK_E // 2
    off = part * nb
    in_specs = [
        pl.BlockSpec((BLK_G, D_HID), lambda i: (i, 0)),
        pl.BlockSpec((BLK_E, D_EDGE), lambda i: (i + off, 0)),
        pl.BlockSpec((D_EDGE, D_HID), lambda i: (0, 0)),
        pl.BlockSpec((D_HID, D_HID), lambda i: (0, 0)),
        pl.BlockSpec((1, D_HID), lambda i: (0, 0)),
    ]
    args = (g, he, we1c, we2, be2)
    body = _edge_body
    aliases = {}
    if part == 1:
        in_specs.append(pl.BlockSpec(memory_space=pltpu.MemorySpace.HBM))
        args = args + (prev,)
        body = _edge_body_p1
        aliases = {5: 0}
    return pl.pallas_call(
        body,
        grid=(nb,),
        in_specs=in_specs,
        out_specs=pl.BlockSpec((BLK_E, D_HID), lambda i: (i + off, 0)),
        out_shape=jax.ShapeDtypeStruct((E, D_HID), jnp.float32),
        input_output_aliases=aliases,
    )(*args)


def _scatter_body(dst_hbm, hen_hbm, agg_hbm,
                  ia, bufa, sa, ib, bufb, sb, agg_sh):
    cid = lax.axis_index("c")
    sid = lax.axis_index("s")
    w = sid * NC + cid
    slots = ((ia, bufa, sa), (ib, bufb, sb))

    zeros = jnp.zeros((L,), jnp.float32)

    def zrow(r, carry):
        for j in range(D_HID // L):
            bufa[r, pl.ds(j * L, L)] = zeros
        return carry

    lax.fori_loop(0, ZROWS, zrow, 0)
    for k in range(ROWS_PER_TILE // ZROWS):
        pltpu.sync_copy(bufa, agg_sh.at[pl.ds(sid * ROWS_PER_TILE + k * ZROWS, ZROWS)])
    plsc.subcore_barrier()

    def start(i, slot):
        idxv, buf, sem = slot
        c = i * NW + w

        @pl.when(c < NCH)
        def _():
            base = c * CHUNK
            pltpu.sync_copy(dst_hbm.at[pl.ds(base, CHUNK)], idxv)
            pltpu.async_copy(hen_hbm.at[pl.ds(base, CHUNK)], buf, sem)

    def finish(i, slot):
        idxv, buf, sem = slot
        c = i * NW + w

        @pl.when(c < NCH)
        def _():
            base = c * CHUNK
            pltpu.make_async_copy(hen_hbm.at[pl.ds(base, CHUNK)], buf, sem).wait()
            pltpu.sync_copy(buf, agg_sh.at[idxv], add=True)

    for b in range(2):
        start(b, slots[b])

    def pair(p, carry):
        for b in range(2):
            i = p * 2 + b
            finish(i, slots[b])
            start(i + 2, slots[b])
        return carry

    lax.fori_loop(0, (ITERS + 1) // 2, pair, 0)
    plsc.subcore_barrier()

    for k in range(ROWS_PER_TILE // ZROWS):
        r0 = sid * ROWS_PER_TILE + k * ZROWS
        pltpu.sync_copy(agg_sh.at[pl.ds(r0, ZROWS)], agg_hbm.at[cid, pl.ds(r0, ZROWS)])


def _scatter(dst, he_new):
    slot = [
        pltpu.VMEM((CHUNK,), jnp.int32),
        pltpu.VMEM((CHUNK, D_HID), jnp.float32),
        pltpu.SemaphoreType.DMA,
    ]
    fn = pl.kernel(
        _scatter_body,
        out_type=jax.ShapeDtypeStruct((NC, N_PAD, D_HID), jnp.float32),
        mesh=plsc.VectorSubcoreMesh(
            core_axis_name="c", subcore_axis_name="s",
            num_cores=NC, num_subcores=NS),
        scratch_types=slot + slot + [pltpu.VMEM_SHARED((N_PAD, D_HID), jnp.float32)],
    )
    return fn(dst, he_new)


def _node_body(hn_ref, a0_ref, a1_ref, wn1a_ref, wn1b_ref, bn1_ref,
               wn2_ref, bn2_ref, wp_ref, bp_ref, out_ref):
    agg = a0_ref[...] + a1_ref[...]
    u = (jnp.dot(hn_ref[...], wn1a_ref[...], preferred_element_type=jnp.float32)
         + jnp.dot(agg, wn1b_ref[...], preferred_element_type=jnp.float32)
         + bn1_ref[...])
    u = u * jax.nn.sigmoid(u)
    v = jnp.dot(u, wn2_ref[...], preferred_element_type=jnp.float32) + bn2_ref[...]
    out_ref[...] = jnp.dot(v, wp_ref[...], preferred_element_type=jnp.float32) + bp_ref[...]


def _node_mlp(hn, a0, a1, wn1a, wn1b, bn1, wn2, bn2, wp, bp):
    blk = 1000
    return pl.pallas_call(
        _node_body,
        grid=(N // blk,),
        in_specs=[
            pl.BlockSpec((blk, D_LAT), lambda i: (i, 0)),
            pl.BlockSpec((blk, D_HID), lambda i: (i, 0)),
            pl.BlockSpec((blk, D_HID), lambda i: (i, 0)),
            pl.BlockSpec((D_LAT, D_HID), lambda i: (0, 0)),
            pl.BlockSpec((D_HID, D_HID), lambda i: (0, 0)),
            pl.BlockSpec((1, D_HID), lambda i: (0, 0)),
            pl.BlockSpec((D_HID, D_HID), lambda i: (0, 0)),
            pl.BlockSpec((1, D_HID), lambda i: (0, 0)),
            pl.BlockSpec((D_HID, D_OUT), lambda i: (0, 0)),
            pl.BlockSpec((1, D_OUT), lambda i: (0, 0)),
        ],
        out_specs=pl.BlockSpec((blk, D_OUT), lambda i: (i, 0)),
        out_shape=jax.ShapeDtypeStruct((N, D_OUT), jnp.float32),
    )(hn, a0, a1, wn1a, wn1b, bn1, wn2, bn2, wp, bp)


def kernel(hn, he, edge_index, We1, be1, We2, be2, Wn1, bn1, Wn2, bn2, Wp, bp):
    src = edge_index[0]
    dst = edge_index[1]
    we1a = We1[:D_LAT]
    we1b = We1[D_LAT:2 * D_LAT]
    we1c = We1[2 * D_LAT:]

    tpk = _precompute(hn, we1a, we1b, be1.reshape(1, -1))
    g1 = _gather(src, dst, tpk, 0, NCH // 2)
    g2 = _gather(src, dst, tpk, NCH // 2, NCH)
    b2r = be2.reshape(1, -1)
    h1 = _edge_mlp(g1, he, we1c, We2, b2r, 0)
    he_new = _edge_mlp(g2, he, we1c, We2, b2r, 1, prev=h1)
    aggs = _scatter(dst, he_new)
    hn_out = _node_mlp(hn, aggs[0], aggs[1],
                       Wn1[:D_LAT], Wn1[D_LAT:], bn1.reshape(1, -1),
                       Wn2, bn2.reshape(1, -1), Wp, bp.reshape(1, -1))
    return hn_out, he_new


# speedup vs baseline: 1.8378x; 1.1051x over previous
import jax
import jax.numpy as jnp
from jax import lax
from jax.experimental import pallas as pl
from jax.experimental.pallas import tpu as pltpu
from jax.experimental.pallas import tpu_sc as plsc

N = 10000
E = 320000
D_LAT = 128
D_EDGE = 16
D_HID = 128
D_OUT = 64

NC = 2
NS = 16
NW = NC * NS
L = 16

CHUNK = 128
HCH = CHUNK // 2
NCH = E // CHUNK
ITERS = (NCH + NW - 1) // NW
ITERS2 = (NCH + NS - 1) // NS
N_PAD = 10240
NHALF = N_PAD // 2
TROWS_T = NHALF // NS
ROWS_PER_TILE = N_PAD // NS
ZROWS = CHUNK


def _pre_body(hn_ref, wa_ref, wb_ref, b1_ref, t_ref):
    h = hn_ref[...]
    p1 = jnp.dot(h, wa_ref[...], preferred_element_type=jnp.float32) + b1_ref[...]
    p2 = jnp.dot(h, wb_ref[...], preferred_element_type=jnp.float32)
    u1 = jax.lax.bitcast_convert_type(p1, jnp.uint32)
    u2 = jax.lax.bitcast_convert_type(p2, jnp.uint32)
    r1 = (u1 + jnp.uint32(0x8000)) >> 16
    r2 = (u2 + jnp.uint32(0x8000)) & jnp.uint32(0xFFFF0000)
    t_ref[...] = jax.lax.bitcast_convert_type(r1 | r2, jnp.float32)


def _precompute(hn, we1a, we1b, be1):
    blk = 1000
    return pl.pallas_call(
        _pre_body,
        grid=(N // blk,),
        in_specs=[
            pl.BlockSpec((blk, D_LAT), lambda i: (i, 0)),
            pl.BlockSpec((D_LAT, D_HID), lambda i: (0, 0)),
            pl.BlockSpec((D_LAT, D_HID), lambda i: (0, 0)),
            pl.BlockSpec((1, D_HID), lambda i: (0, 0)),
        ],
        out_specs=pl.BlockSpec((blk, D_HID), lambda i: (i, 0)),
        out_shape=jax.ShapeDtypeStruct((N, D_HID), jnp.float32),
    )(hn, we1a, we1b, be1)


def _make_gather_body(c0, c1):
    iters = (c1 - c0 + NW - 1) // NW

    def _gather_body(src_hbm, dst_hbm, t_hbm, g_hbm,
                     i1a, i2a, b1a, b2a, bga, sga,
                     i1b, i2b, b1b, b2b, bgb, sgb):
        cid = lax.axis_index("c")
        sid = lax.axis_index("s")
        w = sid * NC + cid

        slots = ((i1a, i2a, b1a, b2a, bga, sga), (i1b, i2b, b1b, b2b, bgb, sgb))

        def start(i, slot):
            i1, i2, b1, b2, bg, sg = slot
            c = c0 + i * NW + w

            @pl.when(c < c1)
            def _():
                base = c * CHUNK
                pltpu.sync_copy(src_hbm.at[pl.ds(base, CHUNK)], i1)
                pltpu.sync_copy(dst_hbm.at[pl.ds(base, CHUNK)], i2)
                pltpu.async_copy(t_hbm.at[i1], b1, sg)
                pltpu.async_copy(t_hbm.at[i2], b2, sg)

        def finish(i, slot):
            i1, i2, b1, b2, bg, sg = slot
            c = c0 + i * NW + w

            @pl.when(c < c1)
            def _():
                pltpu.make_async_copy(t_hbm.at[i1], b1, sg).wait()
                pltpu.make_async_copy(t_hbm.at[i2], b2, sg).wait()

                M = jnp.uint32(0xFFFF0000)
                R = jnp.uint32(0x8000)

                def colsum(w1, w2):
                    u1 = jax.lax.bitcast_convert_type(w1, jnp.uint32)
                    u2 = jax.lax.bitcast_convert_type(w2, jnp.uint32)
                    return (jax.lax.bitcast_convert_type(u1 << 16, jnp.float32)
                            + jax.lax.bitcast_convert_type(u2 & M, jnp.float32))

                def pairrow(t, rcarry):
                    rb = t + HCH
                    for j in range(D_HID // L):
                        cs = pl.ds(j * L, L)
                        a = colsum(b1[t, cs], b2[t, cs])
                        b = colsum(b1[rb, cs], b2[rb, cs])
                        ua = (jax.lax.bitcast_convert_type(a, jnp.uint32) + R) >> 16
                        ub = (jax.lax.bitcast_convert_type(b, jnp.uint32) + R) & M
                        bg[t, cs] = jax.lax.bitcast_convert_type(ua | ub, jnp.float32)
                    return rcarry

                lax.fori_loop(0, HCH, pairrow, 0)
                pltpu.sync_copy(bg, g_hbm.at[pl.ds((c - c0) * HCH, HCH)])

        for b in range(2):
            start(b, slots[b])

        def pair(p, carry):
            for b in range(2):
                i = p * 2 + b
                finish(i, slots[b])
                start(i + 2, slots[b])
            return carry

        lax.fori_loop(0, (iters + 1) // 2, pair, 0)

    return _gather_body


def _gather(src, dst, tpk, c0, c1):
    slot = [
        pltpu.VMEM((CHUNK,), jnp.int32),
        pltpu.VMEM((CHUNK,), jnp.int32),
        pltpu.VMEM((CHUNK, D_HID), jnp.float32),
        pltpu.VMEM((CHUNK, D_HID), jnp.float32),
        pltpu.VMEM((HCH, D_HID), jnp.float32),
        pltpu.SemaphoreType.DMA,
    ]
    fn = pl.kernel(
        _make_gather_body(c0, c1),
        out_type=jax.ShapeDtypeStruct(((c1 - c0) * HCH, D_HID), jnp.float32),
        mesh=plsc.VectorSubcoreMesh(
            core_axis_name="c", subcore_axis_name="s",
            num_cores=NC, num_subcores=NS),
        scratch_types=slot + slot,
    )
    return fn(src, dst, tpk)


KPB = 20
BLK_E = KPB * CHUNK
BLK_G = KPB * HCH


def _unpack(g):
    u = jax.lax.bitcast_convert_type(g, jnp.uint32)
    lo = jax.lax.bitcast_convert_type(u << 16, jnp.float32)
    hi = jax.lax.bitcast_convert_type(u & jnp.uint32(0xFFFF0000), jnp.float32)
    return lo, hi


def _edge_body(g_ref, he_ref, wc_ref, w2_ref, b2_ref, out_ref):
    lo, hi = _unpack(g_ref[...])
    he_blk = he_ref[...]
    hea = jnp.concatenate(
        [he_blk[CHUNK * k:CHUNK * k + HCH] for k in range(KPB)], axis=0)
    heb = jnp.concatenate(
        [he_blk[CHUNK * k + HCH:CHUNK * (k + 1)] for k in range(KPB)], axis=0)
    xa = lo + jnp.dot(hea, wc_ref[...], preferred_element_type=jnp.float32)
    xb = hi + jnp.dot(heb, wc_ref[...], preferred_element_type=jnp.float32)
    sa = xa * jax.nn.sigmoid(xa)
    sb = xb * jax.nn.sigmoid(xb)
    oa = jnp.dot(sa, w2_ref[...], preferred_element_type=jnp.float32) + b2_ref[...]
    ob = jnp.dot(sb, w2_ref[...], preferred_element_type=jnp.float32) + b2_ref[...]
    for k in range(KPB):
        out_ref[pl.ds(CHUNK * k, HCH), :] = oa[HCH * k:HCH * (k + 1)]
        out_ref[pl.ds(CHUNK * k + HCH, HCH), :] = ob[HCH * k:HCH * (k + 1)]


def _edge_body_p1(g_ref, he_ref, wc_ref, w2_ref, b2_ref, prev_ref, out_ref):
    _edge_body(g_ref, he_ref, wc_ref, w2_ref, b2_ref, out_ref)


def _edge_mlp(g, he, we1c, we2, be2, part, prev=None):
    nb = E // BLK_E // 2
    off = part * nb
    in_specs = [
        pl.BlockSpec((BLK_G, D_HID), lambda i: (i, 0)),
        pl.BlockSpec((BLK_E, D_EDGE), lambda i: (i + off, 0)),
        pl.BlockSpec((D_EDGE, D_HID), lambda i: (0, 0)),
        pl.BlockSpec((D_HID, D_HID), lambda i: (0, 0)),
        pl.BlockSpec((1, D_HID), lambda i: (0, 0)),
    ]
    args = (g, he, we1c, we2, be2)
    body = _edge_body
    aliases = {}
    if part == 1:
        in_specs.append(pl.BlockSpec(memory_space=pltpu.MemorySpace.HBM))
        args = args + (prev,)
        body = _edge_body_p1
        aliases = {5: 0}
    return pl.pallas_call(
        body,
        grid=(nb,),
        in_specs=in_specs,
        out_specs=pl.BlockSpec((BLK_E, D_HID), lambda i: (i + off, 0)),
        out_shape=jax.ShapeDtypeStruct((E, D_HID), jnp.float32),
        input_output_aliases=aliases,
    )(*args)


def _scatter_body(dst_hbm, hen_hbm, agg_hbm,
                  ia, bufa, sa, ib, bufb, sb, agg_sh):
    cid = lax.axis_index("c")
    sid = lax.axis_index("s")
    w = sid * NC + cid
    slots = ((ia, bufa, sa), (ib, bufb, sb))

    zeros = jnp.zeros((L,), jnp.float32)

    def zrow(r, carry):
        for j in range(D_HID // L):
            bufa[r, pl.ds(j * L, L)] = zeros
        return carry

    lax.fori_loop(0, ZROWS, zrow, 0)
    for k in range(ROWS_PER_TILE // ZROWS):
        pltpu.sync_copy(bufa, agg_sh.at[pl.ds(sid * ROWS_PER_TILE + k * ZROWS, ZROWS)])
    plsc.subcore_barrier()

    def start(i, slot):
        idxv, buf, sem = slot
        c = i * NW + w

        @pl.when(c < NCH)
        def _():
            base = c * CHUNK
            pltpu.sync_copy(dst_hbm.at[pl.ds(base, CHUNK)], idxv)
            pltpu.async_copy(hen_hbm.at[pl.ds(base, CHUNK)], buf, sem)

    def finish(i, slot):
        idxv, buf, sem = slot
        c = i * NW + w

        @pl.when(c < NCH)
        def _():
            base = c * CHUNK
            pltpu.make_async_copy(hen_hbm.at[pl.ds(base, CHUNK)], buf, sem).wait()
            pltpu.sync_copy(buf, agg_sh.at[idxv], add=True)

    for b in range(2):
        start(b, slots[b])

    def pair(p, carry):
        for b in range(2):
            i = p * 2 + b
            finish(i, slots[b])
            start(i + 2, slots[b])
        return carry

    lax.fori_loop(0, (ITERS + 1) // 2, pair, 0)
    plsc.subcore_barrier()

    for k in range(ROWS_PER_TILE // ZROWS):
        r0 = sid * ROWS_PER_TILE + k * ZROWS
        pltpu.sync_copy(agg_sh.at[pl.ds(r0, ZROWS)], agg_hbm.at[cid, pl.ds(r0, ZROWS)])


def _scatter(dst, he_new):
    slot = [
        pltpu.VMEM((CHUNK,), jnp.int32),
        pltpu.VMEM((CHUNK, D_HID), jnp.float32),
        pltpu.SemaphoreType.DMA,
    ]
    fn = pl.kernel(
        _scatter_body,
        out_type=jax.ShapeDtypeStruct((NC, N_PAD, D_HID), jnp.float32),
        mesh=plsc.VectorSubcoreMesh(
            core_axis_name="c", subcore_axis_name="s",
            num_cores=NC, num_subcores=NS),
        scratch_types=slot + slot + [pltpu.VMEM_SHARED((N_PAD, D_HID), jnp.float32)],
    )
    return fn(dst, he_new)


def _node_body(hn_ref, a0_ref, a1_ref, wn1a_ref, wn1b_ref, bn1_ref,
               wn2_ref, bn2_ref, wp_ref, bp_ref, out_ref):
    agg = a0_ref[...] + a1_ref[...]
    u = (jnp.dot(hn_ref[...], wn1a_ref[...], preferred_element_type=jnp.float32)
         + jnp.dot(agg, wn1b_ref[...], preferred_element_type=jnp.float32)
         + bn1_ref[...])
    u = u * jax.nn.sigmoid(u)
    v = jnp.dot(u, wn2_ref[...], preferred_element_type=jnp.float32) + bn2_ref[...]
    out_ref[...] = jnp.dot(v, wp_ref[...], preferred_element_type=jnp.float32) + bp_ref[...]


def _node_mlp(hn, a0, a1, wn1a, wn1b, bn1, wn2, bn2, wp, bp):
    blk = 1000
    return pl.pallas_call(
        _node_body,
        grid=(N // blk,),
        in_specs=[
            pl.BlockSpec((blk, D_LAT), lambda i: (i, 0)),
            pl.BlockSpec((blk, D_HID), lambda i: (i, 0)),
            pl.BlockSpec((blk, D_HID), lambda i: (i, 0)),
            pl.BlockSpec((D_LAT, D_HID), lambda i: (0, 0)),
            pl.BlockSpec((D_HID, D_HID), lambda i: (0, 0)),
            pl.BlockSpec((1, D_HID), lambda i: (0, 0)),
            pl.BlockSpec((D_HID, D_HID), lambda i: (0, 0)),
            pl.BlockSpec((1, D_HID), lambda i: (0, 0)),
            pl.BlockSpec((D_HID, D_OUT), lambda i: (0, 0)),
            pl.BlockSpec((1, D_OUT), lambda i: (0, 0)),
        ],
        out_specs=pl.BlockSpec((blk, D_OUT), lambda i: (i, 0)),
        out_shape=jax.ShapeDtypeStruct((N, D_OUT), jnp.float32),
    )(hn, a0, a1, wn1a, wn1b, bn1, wn2, bn2, wp, bp)


def kernel(hn, he, edge_index, We1, be1, We2, be2, Wn1, bn1, Wn2, bn2, Wp, bp):
    src = edge_index[0]
    dst = edge_index[1]
    we1a = We1[:D_LAT]
    we1b = We1[D_LAT:2 * D_LAT]
    we1c = We1[2 * D_LAT:]

    tpk = _precompute(hn, we1a, we1b, be1.reshape(1, -1))
    g1 = _gather(src, dst, tpk, 0, NCH // 2)
    g2 = _gather(src, dst, tpk, NCH // 2, NCH)
    b2r = be2.reshape(1, -1)
    h1 = _edge_mlp(g1, he, we1c, We2, b2r, 0)
    he_new = _edge_mlp(g2, he, we1c, We2, b2r, 1, prev=h1)
    aggs = _scatter(dst, he_new)
    hn_out = _node_mlp(hn, aggs[0], aggs[1],
                       Wn1[:D_LAT], Wn1[D_LAT:], bn1.reshape(1, -1),
                       Wn2, bn2.reshape(1, -1), Wp, bp.reshape(1, -1))
    return hn_out, he_new
